# Initial kernel scaffold; baseline (speedup 1.0000x reference)
import jax
import jax.numpy as jnp
from jax import lax
from jax.experimental import pallas as pl
from jax.experimental.pallas import tpu as pltpu
from jax.experimental.pallas import tpu_sc as plsc

N = 10000
D = 128
E = 320000
NC, NS = 2, 16
NW = NC * NS
EPT = E // NW
CHUNK = 80
NCHUNK = EPT // CHUNK
RPT = N // NS


def _proj_body(x_ref, w_ref, ap_ref, y_ref, s_ref):
    y = jnp.dot(x_ref[...], w_ref[...], preferred_element_type=jnp.float32,
                precision=lax.Precision.HIGHEST)
    y_ref[...] = y
    s_ref[...] = jnp.dot(y, ap_ref[...], preferred_element_type=jnp.float32,
                         precision=lax.Precision.HIGHEST)


_PROJ_BM = 2000
_proj_call = pl.pallas_call(
    _proj_body,
    grid=(N // _PROJ_BM,),
    in_specs=[
        pl.BlockSpec((_PROJ_BM, D), lambda i: (i, 0)),
        pl.BlockSpec((D, D), lambda i: (0, 0)),
        pl.BlockSpec((D, 8), lambda i: (0, 0)),
    ],
    out_specs=[
        pl.BlockSpec((_PROJ_BM, D), lambda i: (i, 0)),
        pl.BlockSpec((_PROJ_BM, 8), lambda i: (i, 0)),
    ],
    out_shape=[
        jax.ShapeDtypeStruct((N, D), jnp.float32),
        jax.ShapeDtypeStruct((N, 8), jnp.float32),
    ],
)


def _edge_body(y_hbm, s1_hbm, s2_hbm, row_hbm, col_hbm,
               agg_out, den_out,
               row_v, col_v, s1_v, s2_v, ee_v, rows_v, zden_v,
               agg_sh, den_sh, sem):
    cid = lax.axis_index("c")
    sid = lax.axis_index("s")
    wid = cid * NS + sid

    pltpu.sync_copy(row_hbm.at[wid], row_v)
    pltpu.sync_copy(col_hbm.at[wid], col_v)
    pltpu.sync_copy(s1_hbm, s1_v)
    pltpu.sync_copy(s2_hbm, s2_v)

    z16 = jnp.zeros((16,), jnp.float32)

    def _zero_rows(i, carry):
        for v in range(D // 16):
            rows_v[i, pl.ds(v * 16, 16)] = z16
        return carry

    lax.fori_loop(0, CHUNK, _zero_rows, 0)

    def _zero_zden(i, carry):
        zden_v[pl.ds(i * 16, 16)] = z16
        return carry

    lax.fori_loop(0, 63, _zero_zden, 0)

    base = sid * RPT
    for t in range(RPT // CHUNK):
        pltpu.sync_copy(rows_v, agg_sh.at[pl.ds(base + t * CHUNK, CHUNK)])
    rem = RPT - (RPT // CHUNK) * CHUNK
    pltpu.sync_copy(rows_v.at[pl.ds(0, rem)],
                    agg_sh.at[pl.ds(base + RPT - rem, rem)])

    @pl.when(sid < 10)
    def _():
        pltpu.sync_copy(zden_v.at[pl.ds(0, 1000)],
                        den_sh.at[pl.ds(sid * 1000, 1000)])

    plsc.subcore_barrier()

    def _chunk(j, carry):
        for i in range(CHUNK // 16):
            rv = row_v[j, pl.ds(i * 16, 16)]
            cv = col_v[j, pl.ds(i * 16, 16)]
            e = plsc.load_gather(s1_v, [rv]) + plsc.load_gather(s2_v, [cv])
            e = jnp.where(e >= 0.0, e, 0.2 * e)
            ee_v[pl.ds(i * 16, 16)] = jnp.exp(e)
        pltpu.sync_copy(ee_v, den_sh.at[row_v.at[j]], add=True)
        pltpu.async_copy(y_hbm.at[col_v.at[j]], rows_v, sem).wait()

        def _scale(i, c2):
            aa = ee_v[i]
            for v in range(D // 16):
                sl = pl.ds(v * 16, 16)
                rows_v[i, sl] = rows_v[i, sl] * aa
            return c2

        lax.fori_loop(0, CHUNK, _scale, 0)
        pltpu.sync_copy(rows_v, agg_sh.at[row_v.at[j]], add=True)
        return carry

    lax.fori_loop(0, NCHUNK, _chunk, 0)

    plsc.subcore_barrier()

    pltpu.sync_copy(agg_sh.at[pl.ds(base, RPT)],
                    agg_out.at[cid, pl.ds(base, RPT)])

    @pl.when(sid == 0)
    def _():
        pltpu.sync_copy(den_sh, den_out.at[cid])


_edge_call = pl.kernel(
    _edge_body,
    out_type=(
        jax.ShapeDtypeStruct((NC, N, D), jnp.float32),
        jax.ShapeDtypeStruct((NC, N), jnp.float32),
    ),
    mesh=plsc.VectorSubcoreMesh(core_axis_name="c", subcore_axis_name="s"),
    scratch_types=[
        pltpu.VMEM((NCHUNK, CHUNK), jnp.int32),
        pltpu.VMEM((NCHUNK, CHUNK), jnp.int32),
        pltpu.VMEM((N,), jnp.float32),
        pltpu.VMEM((N,), jnp.float32),
        pltpu.VMEM((CHUNK,), jnp.float32),
        pltpu.VMEM((CHUNK, D), jnp.float32),
        pltpu.VMEM((1008,), jnp.float32),
        pltpu.VMEM_SHARED((N, D), jnp.float32),
        pltpu.VMEM_SHARED((N,), jnp.float32),
        pltpu.SemaphoreType.DMA,
    ],
)


def _final_body(agg_ref, den_ref, y_ref, o_ref):
    d = den_ref[0] + den_ref[1]
    d = jnp.where(d > 0.0, d, 1.0)
    x = (agg_ref[0] + agg_ref[1]) / d + y_ref[...]
    o_ref[...] = jnp.where(x > 0.0, x, jnp.exp(x) - 1.0)


_FIN_BM = 1250
_final_call = pl.pallas_call(
    _final_body,
    grid=(N // _FIN_BM,),
    in_specs=[
        pl.BlockSpec((NC, _FIN_BM, D), lambda i: (0, i, 0)),
        pl.BlockSpec((NC, _FIN_BM, 1), lambda i: (0, i, 0)),
        pl.BlockSpec((_FIN_BM, D), lambda i: (i, 0)),
    ],
    out_specs=pl.BlockSpec((_FIN_BM, D), lambda i: (i, 0)),
    out_shape=jax.ShapeDtypeStruct((N, D), jnp.float32),
)


@jax.jit
def kernel(input, triple, W, a):
    row3 = triple[:, 0].astype(jnp.int32).reshape(NW, NCHUNK, CHUNK)
    col3 = triple[:, 2].astype(jnp.int32).reshape(NW, NCHUNK, CHUNK)
    a_pad = jnp.zeros((D, 8), jnp.float32)
    a_pad = a_pad.at[:, 0].set(a[:D, 0]).at[:, 1].set(a[D:, 0])

    y, s12 = _proj_call(input.astype(jnp.float32), W.astype(jnp.float32), a_pad)
    s1 = jnp.ascontiguousarray(s12[:, 0])
    s2 = jnp.ascontiguousarray(s12[:, 1])

    agg2, den2 = _edge_call(y, s1, s2, row3, col3)
    return _final_call(agg2, den2.reshape(NC, N, 1), y)


# speedup vs baseline: 17.4224x; 17.4224x over previous
import jax
import jax.numpy as jnp
from jax import lax
from jax.experimental import pallas as pl
from jax.experimental.pallas import tpu as pltpu
from jax.experimental.pallas import tpu_sc as plsc

N = 10000
D = 128
E = 320000
NC, NS = 2, 16
NW = NC * NS
EPT = E // NW
CHUNK = 80
NCHUNK = EPT // CHUNK
RPT = 624


def _proj_body(x_ref, w_ref, ap_ref, y_ref, s_ref):
    y = jnp.dot(x_ref[...], w_ref[...], preferred_element_type=jnp.float32,
                precision=lax.Precision.HIGHEST)
    y_ref[...] = y
    s_ref[...] = jnp.dot(y, ap_ref[...], preferred_element_type=jnp.float32,
                         precision=lax.Precision.HIGHEST)


_PROJ_BM = 2000
_proj_call = pl.pallas_call(
    _proj_body,
    grid=(N // _PROJ_BM,),
    in_specs=[
        pl.BlockSpec((_PROJ_BM, D), lambda i: (i, 0)),
        pl.BlockSpec((D, D), lambda i: (0, 0)),
        pl.BlockSpec((D, 8), lambda i: (0, 0)),
    ],
    out_specs=[
        pl.BlockSpec((_PROJ_BM, D), lambda i: (i, 0)),
        pl.BlockSpec((_PROJ_BM, 8), lambda i: (i, 0)),
    ],
    out_shape=[
        jax.ShapeDtypeStruct((N, D), jnp.float32),
        jax.ShapeDtypeStruct((N, 8), jnp.float32),
    ],
)


def _edge_body(y_hbm, s1_hbm, s2_hbm, row_hbm, col_hbm,
               agg_out, den_out,
               row_c, col_c, s1_v, s2_v, ee_v, rows_v, zden_v,
               agg_sh, den_sh, sem):
    cid = lax.axis_index("c")
    sid = lax.axis_index("s")
    wid = cid * NS + sid

    pltpu.sync_copy(s1_hbm, s1_v)
    pltpu.sync_copy(s2_hbm, s2_v)

    z16 = jnp.zeros((16,), jnp.float32)

    def _zero_rows(i, carry):
        for v in range(D // 16):
            rows_v[i, pl.ds(v * 16, 16)] = z16
        return carry

    lax.fori_loop(0, CHUNK, _zero_rows, 0)

    def _zero_zden(i, carry):
        zden_v[pl.ds(i * 16, 16)] = z16
        return carry

    lax.fori_loop(0, 63, _zero_zden, 0)

    base = sid * RPT
    for t in range(RPT // CHUNK):
        pltpu.sync_copy(rows_v, agg_sh.at[pl.ds(base + t * CHUNK, CHUNK)])
    rem = RPT - (RPT // CHUNK) * CHUNK
    pltpu.sync_copy(rows_v.at[pl.ds(0, rem)],
                    agg_sh.at[pl.ds(base + RPT - rem, rem)])

    @pl.when(sid < 2)
    def _():
        pltpu.sync_copy(rows_v.at[pl.ds(0, 8)],
                        agg_sh.at[pl.ds(NS * RPT + sid * 8, 8)])

    @pl.when(sid < 10)
    def _():
        pltpu.sync_copy(zden_v.at[pl.ds(0, 1000)],
                        den_sh.at[pl.ds(sid * 1000, 1000)])

    plsc.subcore_barrier()

    def _chunk(j, carry):
        pltpu.sync_copy(row_hbm.at[wid, j], row_c)
        pltpu.sync_copy(col_hbm.at[wid, j], col_c)
        for i in range(CHUNK // 16):
            rv = row_c[pl.ds(i * 16, 16)]
            cv = col_c[pl.ds(i * 16, 16)]
            e = plsc.load_gather(s1_v, [rv]) + plsc.load_gather(s2_v, [cv])
            e = jnp.where(e >= 0.0, e, 0.2 * e)
            ee_v[pl.ds(i * 16, 16)] = jnp.exp(e)
        pltpu.sync_copy(ee_v, den_sh.at[row_c], add=True)
        pltpu.async_copy(y_hbm.at[col_c], rows_v, sem).wait()

        def _scale(g, c2):
            eev = ee_v[pl.ds(g * 16, 16)]
            for l in range(16):
                aa = eev[l]
                i = g * 16 + l
                for v in range(D // 16):
                    sl = pl.ds(v * 16, 16)
                    rows_v[i, sl] = rows_v[i, sl] * aa
            return c2

        lax.fori_loop(0, CHUNK // 16, _scale, 0)
        pltpu.sync_copy(rows_v, agg_sh.at[row_c], add=True)
        return carry

    lax.fori_loop(0, NCHUNK, _chunk, 0)

    plsc.subcore_barrier()

    pltpu.sync_copy(agg_sh.at[pl.ds(base, RPT)],
                    agg_out.at[cid, pl.ds(base, RPT)])

    @pl.when(sid < 2)
    def _():
        pltpu.sync_copy(agg_sh.at[pl.ds(NS * RPT + sid * 8, 8)],
                        agg_out.at[cid, pl.ds(NS * RPT + sid * 8, 8)])

    @pl.when(sid == 0)
    def _():
        pltpu.sync_copy(den_sh, den_out.at[cid])


_edge_call = pl.kernel(
    _edge_body,
    out_type=(
        jax.ShapeDtypeStruct((NC, N, D), jnp.float32),
        jax.ShapeDtypeStruct((NC, N), jnp.float32),
    ),
    mesh=plsc.VectorSubcoreMesh(core_axis_name="c", subcore_axis_name="s"),
    compiler_params=pltpu.CompilerParams(needs_layout_passes=False),
    scratch_types=[
        pltpu.VMEM((CHUNK,), jnp.int32),
        pltpu.VMEM((CHUNK,), jnp.int32),
        pltpu.VMEM((N,), jnp.float32),
        pltpu.VMEM((N,), jnp.float32),
        pltpu.VMEM((CHUNK,), jnp.float32),
        pltpu.VMEM((CHUNK, D), jnp.float32),
        pltpu.VMEM((1008,), jnp.float32),
        pltpu.VMEM_SHARED((N, D), jnp.float32),
        pltpu.VMEM_SHARED((N,), jnp.float32),
        pltpu.SemaphoreType.DMA,
    ],
)


def _final_body(agg_ref, den_ref, y_ref, o_ref):
    d = den_ref[0] + den_ref[1]
    d = jnp.where(d > 0.0, d, 1.0)
    x = (agg_ref[0] + agg_ref[1]) / d + y_ref[...]
    o_ref[...] = jnp.where(x > 0.0, x, jnp.exp(x) - 1.0)


_FIN_BM = 2000
_final_call = pl.pallas_call(
    _final_body,
    grid=(N // _FIN_BM,),
    in_specs=[
        pl.BlockSpec((NC, _FIN_BM, D), lambda i: (0, i, 0)),
        pl.BlockSpec((NC, _FIN_BM, 1), lambda i: (0, i, 0)),
        pl.BlockSpec((_FIN_BM, D), lambda i: (i, 0)),
    ],
    out_specs=pl.BlockSpec((_FIN_BM, D), lambda i: (i, 0)),
    out_shape=jax.ShapeDtypeStruct((N, D), jnp.float32),
)


@jax.jit
def kernel(input, triple, W, a):
    row3 = triple[:, 0].astype(jnp.int32).reshape(NW, NCHUNK, CHUNK)
    col3 = triple[:, 2].astype(jnp.int32).reshape(NW, NCHUNK, CHUNK)
    a_pad = jnp.zeros((D, 8), jnp.float32)
    a_pad = a_pad.at[:, 0].set(a[:D, 0]).at[:, 1].set(a[D:, 0])

    y, s12 = _proj_call(input.astype(jnp.float32), W.astype(jnp.float32), a_pad)
    s1 = s12[:, 0] + 0.0
    s2 = s12[:, 1] + 0.0

    agg2, den2 = _edge_call(y, s1, s2, row3, col3)
    return _final_call(agg2, den2.reshape(NC, N, 1), y)
